# Initial kernel scaffold; baseline (speedup 1.0000x reference)
import jax
import jax.numpy as jnp
from jax.experimental import pallas as pl


def kernel(xe, xe_lens, xd, xd_lens, xt, xt_lens, prev):
    raise NotImplementedError("write your pallas kernel here")


# speedup vs baseline: 1.9904x; 1.9904x over previous
import functools

import jax
import jax.numpy as jnp
from jax import lax
from jax.experimental import pallas as pl
from jax.experimental.pallas import tpu as pltpu
from jax.experimental.pallas import tpu_sc as plsc

B = 256
WENC = 4096
WDEC = 4096
MAXS = 2048
NC = 2
NS = 16
NW = NC * NS
ROWS = B // NW
BUF = WENC + 2 * MAXS
L = 16


def _extract(vec, lane):
    m = lax.iota(jnp.int32, L) == lane
    return jnp.sum(jnp.where(m, vec, 0))


def _frames_sc(xe, xe_lens, xd, xt, xt_lens, prev):
    mesh = plsc.VectorSubcoreMesh(core_axis_name="c", subcore_axis_name="s")

    @functools.partial(
        pl.kernel,
        mesh=mesh,
        compiler_params=pltpu.CompilerParams(needs_layout_passes=False),
        out_type=[
            jax.ShapeDtypeStruct((B, WENC), jnp.int32),
            jax.ShapeDtypeStruct((B, WDEC), jnp.int32),
            jax.ShapeDtypeStruct((B, WENC), jnp.int32),
        ],
        scratch_types=[
            pltpu.VMEM((BUF,), jnp.int32),
            pltpu.VMEM((WENC,), jnp.int32),
            pltpu.VMEM((WENC,), jnp.int32),
            pltpu.VMEM((L,), jnp.int32),
            pltpu.VMEM((L,), jnp.int32),
            pltpu.VMEM((MAXS,), jnp.int32),
        ],
    )
    def k(xe_h, el_h, xd_h, xt_h, tl_h, prev_h,
          ye_h, yd_h, p_h, buf_v, ye_v, p_v, el_v, tl_v, zero_v):
        wid = lax.axis_index("s") * NC + lax.axis_index("c")
        base = wid * ROWS
        lanes = lax.iota(jnp.int32, L)

        zv = jnp.zeros((L,), jnp.int32)

        def zero_body(i, carry):
            zero_v[pl.ds(pl.multiple_of(i * L, L), L)] = zv
            return carry

        lax.fori_loop(0, MAXS // L, zero_body, 0)

        pltpu.sync_copy(el_h.at[pl.ds(base, ROWS)], el_v.at[pl.ds(0, ROWS)])
        pltpu.sync_copy(tl_h.at[pl.ds(base, ROWS)], tl_v.at[pl.ds(0, ROWS)])

        el_vec = el_v[...]
        tl_vec = tl_v[...]

        for r in range(ROWS):
            row = base + r
            el = el_vec[r]
            tl = tl_vec[r]
            pltpu.sync_copy(prev_h.at[row], buf_v.at[pl.ds(0, WENC)])
            pltpu.sync_copy(xe_h.at[row], buf_v.at[pl.ds(WENC, MAXS)])
            pltpu.sync_copy(xt_h.at[row], buf_v.at[pl.ds(WENC + MAXS, MAXS)])

            def ye_body(kk, _):
                off = pl.multiple_of(kk * L, L)
                idx = el + off + lanes
                ye_v[pl.ds(off, L)] = plsc.load_gather(buf_v, [idx])
                return 0

            lax.fori_loop(0, WENC // L, ye_body, 0)

            def p_body(kk, _):
                off = pl.multiple_of(kk * L, L)
                q = tl + off + lanes
                idx = q + jnp.where(q < WENC, el, MAXS)
                p_v[pl.ds(off, L)] = plsc.load_gather(buf_v, [idx])
                return 0

            lax.fori_loop(0, WENC // L, p_body, 0)

            pltpu.sync_copy(ye_v, ye_h.at[row])
            pltpu.sync_copy(p_v, p_h.at[row])
            pltpu.sync_copy(xd_h.at[row], yd_h.at[row, pl.ds(0, MAXS)])
            pltpu.sync_copy(zero_v, yd_h.at[row, pl.ds(MAXS, WDEC - MAXS)])

    return k(xe, xe_lens, xd, xt, xt_lens, prev)


def kernel(xe, xe_lens, xd, xd_lens, xt, xt_lens, prev):
    el = xe_lens.astype(jnp.int32)
    dl = xd_lens.astype(jnp.int32)
    ye, yd, p = _frames_sc(xe, el, xd, xt, xt_lens.astype(jnp.int32), prev)
    return (ye, el, yd, dl, p)


# speedup vs baseline: 2.1109x; 1.0606x over previous
import functools

import jax
import jax.numpy as jnp
from jax import lax
from jax.experimental import pallas as pl
from jax.experimental.pallas import tpu as pltpu
from jax.experimental.pallas import tpu_sc as plsc

B = 256
WENC = 4096
WDEC = 4096
MAXS = 2048
NC = 2
NS = 16
NW = NC * NS
ROWS = B // NW
BUF = WENC + 2 * MAXS
L = 16
CHUNKS = WENC // L


def _frames_sc(xe, xe_lens, xd, xt, xt_lens, prev):
    mesh = plsc.VectorSubcoreMesh(core_axis_name="c", subcore_axis_name="s")

    @functools.partial(
        pl.kernel,
        mesh=mesh,
        compiler_params=pltpu.CompilerParams(needs_layout_passes=False),
        out_type=[
            jax.ShapeDtypeStruct((B, WENC), jnp.int32),
            jax.ShapeDtypeStruct((B, WDEC), jnp.int32),
            jax.ShapeDtypeStruct((B, WENC), jnp.int32),
        ],
        scratch_types=[
            pltpu.VMEM((BUF,), jnp.int32),
            pltpu.VMEM((BUF,), jnp.int32),
            pltpu.VMEM((WENC,), jnp.int32),
            pltpu.VMEM((WENC,), jnp.int32),
            pltpu.VMEM((WENC,), jnp.int32),
            pltpu.VMEM((WENC,), jnp.int32),
            pltpu.VMEM((L,), jnp.int32),
            pltpu.VMEM((L,), jnp.int32),
            pltpu.VMEM((MAXS,), jnp.int32),
            pltpu.SemaphoreType.DMA,
            pltpu.SemaphoreType.DMA,
            pltpu.SemaphoreType.DMA,
            pltpu.SemaphoreType.DMA,
            pltpu.SemaphoreType.DMA,
        ],
    )
    def k(xe_h, el_h, xd_h, xt_h, tl_h, prev_h,
          ye_h, yd_h, p_h, buf0, buf1, ye0, ye1, p0, p1, el_v, tl_v, zero_v,
          sem_in0, sem_in1, sem_out0, sem_out1, sem_yd):
        wid = lax.axis_index("s") * NC + lax.axis_index("c")
        base = wid * ROWS
        lanes = lax.iota(jnp.int32, L)
        bufs = (buf0, buf1)
        yes = (ye0, ye1)
        ps = (p0, p1)
        sem_in = (sem_in0, sem_in1)
        sem_out = (sem_out0, sem_out1)

        zv = jnp.zeros((L,), jnp.int32)

        @pl.loop(0, MAXS // L)
        def _(i):
            zero_v[pl.ds(pl.multiple_of(i * L, L), L)] = zv

        pltpu.sync_copy(el_h.at[pl.ds(base, ROWS)], el_v.at[pl.ds(0, ROWS)])
        pltpu.sync_copy(tl_h.at[pl.ds(base, ROWS)], tl_v.at[pl.ds(0, ROWS)])
        el_vec = el_v[...]
        tl_vec = tl_v[...]

        yd_handles = []
        for r in range(ROWS):
            row = base + r
            yd_handles.append(
                pltpu.async_copy(xd_h.at[row], yd_h.at[row, pl.ds(0, MAXS)],
                                 sem_yd))
            yd_handles.append(
                pltpu.async_copy(zero_v, yd_h.at[row, pl.ds(MAXS, WDEC - MAXS)],
                                 sem_yd))

        def start_inputs(r):
            s = sem_in[r % 2]
            buf = bufs[r % 2]
            row = base + r
            return (
                pltpu.async_copy(prev_h.at[row], buf.at[pl.ds(0, WENC)], s),
                pltpu.async_copy(xe_h.at[row], buf.at[pl.ds(WENC, MAXS)], s),
                pltpu.async_copy(xt_h.at[row],
                                 buf.at[pl.ds(WENC + MAXS, MAXS)], s),
            )

        in_flight = {0: start_inputs(0)}
        out_flight = {}
        for r in range(ROWS):
            slot = r % 2
            row = base + r
            if r + 1 < ROWS:
                in_flight[r + 1] = start_inputs(r + 1)
            if r - 2 in out_flight:
                for h in out_flight.pop(r - 2):
                    h.wait()
            for h in in_flight.pop(r):
                h.wait()

            el = el_vec[r]
            tl = tl_vec[r]
            el_lanes = el + lanes
            tl_lanes = tl + lanes
            a_vec = tl_lanes + el
            b_vec = tl_lanes + MAXS
            buf = bufs[slot]
            yev = yes[slot]
            pv = ps[slot]

            @pl.loop(0, CHUNKS)
            def _(kk):
                off = pl.multiple_of(kk * L, L)
                yev[pl.ds(off, L)] = plsc.load_gather(buf, [el_lanes + off])
                q = tl_lanes + off
                idx2 = jnp.where(q < WENC, a_vec + off, b_vec + off)
                pv[pl.ds(off, L)] = plsc.load_gather(buf, [idx2])

            out_flight[r] = (
                pltpu.async_copy(yev, ye_h.at[row], sem_out[slot]),
                pltpu.async_copy(pv, p_h.at[row], sem_out[slot]),
            )

        for r, hs in sorted(out_flight.items()):
            for h in hs:
                h.wait()
        for h in yd_handles:
            h.wait()

    return k(xe, xe_lens, xd, xt, xt_lens, prev)


def kernel(xe, xe_lens, xd, xd_lens, xt, xt_lens, prev):
    el = xe_lens.astype(jnp.int32)
    dl = xd_lens.astype(jnp.int32)
    ye, yd, p = _frames_sc(xe, el, xd, xt, xt_lens.astype(jnp.int32), prev)
    return (ye, el, yd, dl, p)
